# baseline (speedup 1.0000x reference)
import functools

import jax
import jax.numpy as jnp
from jax import lax
from jax.experimental import pallas as pl
from jax.experimental.pallas import tpu as pltpu
from jax.experimental.pallas import tpu_sc as plsc

_TILE_V = 1024
_NBUF = 6


def _sc_gather(table, idx):
    B = idx.shape[0]
    V, D = table.shape
    info = plsc.get_sparse_core_info()
    nc, ns = info.num_cores, info.num_subcores
    nw = nc * ns
    b_per_w = B // nw
    mesh = plsc.VectorSubcoreMesh(core_axis_name="c", subcore_axis_name="s")

    @functools.partial(
        pl.kernel,
        mesh=mesh,
        compiler_params=pltpu.CompilerParams(use_tc_tiling_on_sc=False),
        out_type=jax.ShapeDtypeStruct((B, D), jnp.float32),
        scratch_types=[
            pltpu.VMEM((b_per_w,), jnp.int32),
            pltpu.VMEM((b_per_w, D), jnp.float32),
            pltpu.SemaphoreType.DMA,
        ],
    )
    def gather_kernel(table_hbm, idx_hbm, out_hbm, idx_v, rows_v, sem):
        wid = lax.axis_index("s") * nc + lax.axis_index("c")
        base = wid * b_per_w
        pltpu.sync_copy(idx_hbm.at[pl.ds(base, b_per_w)], idx_v)
        pltpu.async_copy(table_hbm.at[idx_v], rows_v, sem).wait()
        pltpu.sync_copy(rows_v, out_hbm.at[pl.ds(base, b_per_w)])

    return gather_kernel(table, idx)


def _make_mm_body(n_tiles, tile_v, tail_v, nbuf, B):
    def body(e_ref, wt_ref, b_ref, o_hbm, acc_ref, sems):
        j = pl.program_id(0)
        slot = lax.rem(j, nbuf)

        @pl.when(j >= nbuf)
        def _():
            pltpu.make_async_copy(
                acc_ref.at[slot], o_hbm.at[pl.ds(0, tile_v)], sems.at[slot]
            ).wait()

        acc = lax.dot_general(
            wt_ref[...], e_ref[...],
            dimension_numbers=(((0,), (1,)), ((), ())),
            preferred_element_type=jnp.float32,
        )
        acc = acc + lax.dot_general(
            b_ref[...], jnp.ones((1, B), jnp.float32),
            dimension_numbers=(((0,), (0,)), ((), ())),
            preferred_element_type=jnp.float32,
        )
        acc_ref[slot] = acc

        @pl.when(j < n_tiles - 1)
        def _():
            pltpu.make_async_copy(
                acc_ref.at[slot],
                o_hbm.at[pl.ds(j * tile_v, tile_v)],
                sems.at[slot],
            ).start()

        @pl.when(j == n_tiles - 1)
        def _():
            pltpu.make_async_copy(
                acc_ref.at[slot, pl.ds(0, tail_v)],
                o_hbm.at[pl.ds(j * tile_v, tail_v)],
                sems.at[slot],
            ).start()
            for step in range(max(n_tiles - nbuf, 0), n_tiles - 1):
                s = step % nbuf
                pltpu.make_async_copy(
                    acc_ref.at[s], o_hbm.at[pl.ds(0, tile_v)], sems.at[s]
                ).wait()
            pltpu.make_async_copy(
                acc_ref.at[slot, pl.ds(0, tail_v)],
                o_hbm.at[pl.ds(0, tail_v)],
                sems.at[slot],
            ).wait()

    return body


def _projection(embeds, linear_w, linear_b, tile_v=_TILE_V, nbuf=_NBUF):
    B, D = embeds.shape
    V = linear_w.shape[0]
    n_tiles = pl.cdiv(V, tile_v)
    tail_v = V - (n_tiles - 1) * tile_v

    wt = linear_w.T
    bias2d = linear_b.reshape(1, V)

    out_t = pl.pallas_call(
        _make_mm_body(n_tiles, tile_v, tail_v, nbuf, B),
        grid=(n_tiles,),
        in_specs=[
            pl.BlockSpec((B, D), lambda j: (0, 0)),
            pl.BlockSpec((D, tile_v), lambda j: (0, j)),
            pl.BlockSpec((1, tile_v), lambda j: (0, j)),
        ],
        out_specs=pl.BlockSpec(memory_space=pl.ANY),
        out_shape=jax.ShapeDtypeStruct((V, B), jnp.float32),
        scratch_shapes=[
            pltpu.VMEM((nbuf, tile_v, B), jnp.float32),
            pltpu.SemaphoreType.DMA((nbuf,)),
        ],
    )(embeds, wt, bias2d)
    return out_t.T


def kernel(inputs, embedding_table, linear_w, linear_b):
    idx = inputs.astype(jnp.int32)
    embeds = _sc_gather(embedding_table, idx)
    return _projection(embeds, linear_w, linear_b)


# speedup vs baseline: 1.0468x; 1.0468x over previous
import functools

import jax
import jax.numpy as jnp
from jax import lax
from jax.experimental import pallas as pl
from jax.experimental.pallas import tpu as pltpu
from jax.experimental.pallas import tpu_sc as plsc

_TILE_V = 2048
_NBUF = 4


def _sc_gather(table, idx):
    B = idx.shape[0]
    V, D = table.shape
    info = plsc.get_sparse_core_info()
    nc, ns = info.num_cores, info.num_subcores
    nw = nc * ns
    b_per_w = B // nw
    mesh = plsc.VectorSubcoreMesh(core_axis_name="c", subcore_axis_name="s")

    @functools.partial(
        pl.kernel,
        mesh=mesh,
        compiler_params=pltpu.CompilerParams(use_tc_tiling_on_sc=False),
        out_type=jax.ShapeDtypeStruct((B, D), jnp.float32),
        scratch_types=[
            pltpu.VMEM((b_per_w,), jnp.int32),
            pltpu.VMEM((b_per_w, D), jnp.float32),
            pltpu.SemaphoreType.DMA,
        ],
    )
    def gather_kernel(table_hbm, idx_hbm, out_hbm, idx_v, rows_v, sem):
        wid = lax.axis_index("s") * nc + lax.axis_index("c")
        base = wid * b_per_w
        pltpu.sync_copy(idx_hbm.at[pl.ds(base, b_per_w)], idx_v)
        pltpu.async_copy(table_hbm.at[idx_v], rows_v, sem).wait()
        pltpu.sync_copy(rows_v, out_hbm.at[pl.ds(base, b_per_w)])

    return gather_kernel(table, idx)


def _make_mm_body(n_tiles, tile_v, tail_v, nbuf, B):
    def body(e_ref, wt_ref, b_ref, o_hbm, acc_ref, sems):
        j = pl.program_id(0)
        slot = lax.rem(j, nbuf)

        @pl.when(j >= nbuf)
        def _():
            pltpu.make_async_copy(
                acc_ref.at[slot], o_hbm.at[pl.ds(0, tile_v)], sems.at[slot]
            ).wait()

        acc = lax.dot_general(
            wt_ref[...], e_ref[...],
            dimension_numbers=(((0,), (1,)), ((), ())),
            preferred_element_type=jnp.float32,
        )
        acc = acc + lax.dot_general(
            b_ref[...], jnp.ones((1, B), jnp.float32),
            dimension_numbers=(((0,), (0,)), ((), ())),
            preferred_element_type=jnp.float32,
        )
        acc_ref[slot] = acc

        @pl.when(j < n_tiles - 1)
        def _():
            pltpu.make_async_copy(
                acc_ref.at[slot],
                o_hbm.at[pl.ds(j * tile_v, tile_v)],
                sems.at[slot],
            ).start()

        @pl.when(j == n_tiles - 1)
        def _():
            pltpu.make_async_copy(
                acc_ref.at[slot, pl.ds(0, tail_v)],
                o_hbm.at[pl.ds(j * tile_v, tail_v)],
                sems.at[slot],
            ).start()
            for step in range(max(n_tiles - nbuf, 0), n_tiles - 1):
                s = step % nbuf
                pltpu.make_async_copy(
                    acc_ref.at[s], o_hbm.at[pl.ds(0, tile_v)], sems.at[s]
                ).wait()
            pltpu.make_async_copy(
                acc_ref.at[slot, pl.ds(0, tail_v)],
                o_hbm.at[pl.ds(0, tail_v)],
                sems.at[slot],
            ).wait()

    return body


def _projection(embeds, linear_w, linear_b, tile_v=_TILE_V, nbuf=_NBUF):
    B, D = embeds.shape
    V = linear_w.shape[0]
    n_tiles = pl.cdiv(V, tile_v)
    tail_v = V - (n_tiles - 1) * tile_v

    wt = linear_w.T
    bias2d = linear_b.reshape(1, V)

    out_t = pl.pallas_call(
        _make_mm_body(n_tiles, tile_v, tail_v, nbuf, B),
        grid=(n_tiles,),
        in_specs=[
            pl.BlockSpec((B, D), lambda j: (0, 0)),
            pl.BlockSpec((D, tile_v), lambda j: (0, j)),
            pl.BlockSpec((1, tile_v), lambda j: (0, j)),
        ],
        out_specs=pl.BlockSpec(memory_space=pl.ANY),
        out_shape=jax.ShapeDtypeStruct((V, B), jnp.float32),
        scratch_shapes=[
            pltpu.VMEM((nbuf, tile_v, B), jnp.float32),
            pltpu.SemaphoreType.DMA((nbuf,)),
        ],
    )(embeds, wt, bias2d)
    return out_t.T


def kernel(inputs, embedding_table, linear_w, linear_b):
    idx = inputs.astype(jnp.int32)
    embeds = _sc_gather(embedding_table, idx)
    return _projection(embeds, linear_w, linear_b)
